# Initial kernel scaffold; baseline (speedup 1.0000x reference)
import functools

import jax
import jax.numpy as jnp
from jax import lax
from jax.experimental import pallas as pl
from jax.experimental.pallas import tpu as pltpu
from jax.experimental.pallas import tpu_sc as plsc

N = 100000
E = 1600000
G = 256
D = 32
HALF = 16
NC = 2
NS = 16
NW = NC * NS
CH = 128
NCHUNK = E // CH
RPS = N // NS

R = 2000
NBLK = N // R

_f32 = jnp.float32


def _sc_mesh():
    return plsc.VectorSubcoreMesh(core_axis_name="c", subcore_axis_name="s")


def _deg_body(dst_hbm, ones_hbm, zeros_hbm, hist_hbm, dst_v, ones_v, hist_sh):
    c = lax.axis_index("c")
    s = lax.axis_index("s")
    w = s * NC + c
    pltpu.sync_copy(zeros_hbm, hist_sh.at[pl.ds(s * RPS, RPS)])
    pltpu.sync_copy(ones_hbm, ones_v)
    plsc.subcore_barrier()

    def body(i, carry):
        k = w + i * NW

        @pl.when(k < NCHUNK)
        def _():
            pltpu.sync_copy(dst_hbm.at[pl.ds(k * CH, CH)], dst_v)
            pltpu.sync_copy(ones_v, hist_sh.at[dst_v], add=True)

        return carry

    lax.fori_loop(0, (NCHUNK + NW - 1) // NW, body, 0)
    plsc.subcore_barrier()
    pltpu.sync_copy(hist_sh.at[pl.ds(s * RPS, RPS)],
                    hist_hbm.at[pl.ds(c * N + s * RPS, RPS)])


def _deg_hist(dst, ones_ch, zeros_rps):
    return pl.kernel(
        _deg_body,
        out_type=jax.ShapeDtypeStruct((NC * N, HALF), _f32),
        mesh=_sc_mesh(),
        scratch_types=[
            pltpu.VMEM((CH,), jnp.int32),
            pltpu.VMEM((CH, HALF), _f32),
            pltpu.VMEM_SHARED((N, HALF), _f32),
        ],
    )(dst, ones_ch, zeros_rps)


def _conv_body(yflat_hbm, src_hbm, dst_hbm, zeros_hbm, acc_hbm,
               src_v, dst_v, gidx_v, rows_v, sem, acc_sh):
    c = lax.axis_index("c")
    s = lax.axis_index("s")
    pltpu.sync_copy(zeros_hbm, acc_sh.at[pl.ds(s * RPS, RPS)])
    plsc.subcore_barrier()

    def body(i, carry):
        k = s + i * NS

        @pl.when(k < NCHUNK)
        def _():
            e0 = k * CH
            pltpu.sync_copy(src_hbm.at[pl.ds(e0, CH)], src_v)
            pltpu.sync_copy(dst_hbm.at[pl.ds(e0, CH)], dst_v)
            for j in range(CH // 16):
                sl = pl.ds(j * 16, 16)
                gidx_v[sl] = src_v[sl] * 2 + c
            pltpu.async_copy(yflat_hbm.at[gidx_v], rows_v, sem).wait()
            pltpu.sync_copy(rows_v, acc_sh.at[dst_v], add=True)

        return carry

    lax.fori_loop(0, (NCHUNK + NS - 1) // NS, body, 0)
    plsc.subcore_barrier()
    pltpu.sync_copy(acc_sh.at[pl.ds(s * RPS, RPS)],
                    acc_hbm.at[pl.ds(c * N + s * RPS, RPS)])


def _edge_pass(yflat, src, dst, zeros_rps):
    return pl.kernel(
        _conv_body,
        out_type=jax.ShapeDtypeStruct((NC * N, HALF), _f32),
        mesh=_sc_mesh(),
        scratch_types=[
            pltpu.VMEM((CH,), jnp.int32),
            pltpu.VMEM((CH,), jnp.int32),
            pltpu.VMEM((CH,), jnp.int32),
            pltpu.VMEM((CH, HALF), _f32),
            pltpu.SemaphoreType.DMA,
            pltpu.VMEM_SHARED((N, HALF), _f32),
        ],
    )(yflat, src, dst, zeros_rps)


def _m1_body(x_ref, h0_ref, h1_ref, w1_ref, y1_ref, dinv_ref):
    deg = 1.0 + h0_ref[:, :1] + h1_ref[:, :1]
    dinv = lax.rsqrt(deg)
    xw = jnp.dot(x_ref[...], w1_ref[...], preferred_element_type=_f32)
    y1_ref[...] = xw * dinv
    dinv_ref[...] = jnp.broadcast_to(dinv, (R, 8))


def _m1(x, h0, h1, w1):
    return pl.pallas_call(
        _m1_body,
        grid=(NBLK,),
        in_specs=[
            pl.BlockSpec((R, 128), lambda i: (i, 0)),
            pl.BlockSpec((R, HALF), lambda i: (i, 0)),
            pl.BlockSpec((R, HALF), lambda i: (i, 0)),
            pl.BlockSpec((128, D), lambda i: (0, 0)),
        ],
        out_specs=[
            pl.BlockSpec((R, D), lambda i: (i, 0)),
            pl.BlockSpec((R, 8), lambda i: (i, 0)),
        ],
        out_shape=[
            jax.ShapeDtypeStruct((N, D), _f32),
            jax.ShapeDtypeStruct((N, 8), _f32),
        ],
    )(x, h0, h1, w1)


def _k2_body(a0_ref, a1_ref, y1_ref, dinv_ref, w2_ref, b1_ref, y2_ref):
    dinv = dinv_ref[:, :1]
    acc = jnp.concatenate([a0_ref[...], a1_ref[...]], axis=1)
    h1 = jnp.maximum(dinv * (acc + y1_ref[...]) + b1_ref[...], 0.0)
    y2_ref[...] = jnp.dot(h1, w2_ref[...], preferred_element_type=_f32) * dinv


def _k2(a0, a1, y1, dinv, w2p, b1r):
    return pl.pallas_call(
        _k2_body,
        grid=(NBLK,),
        in_specs=[
            pl.BlockSpec((R, HALF), lambda i: (i, 0)),
            pl.BlockSpec((R, HALF), lambda i: (i, 0)),
            pl.BlockSpec((R, D), lambda i: (i, 0)),
            pl.BlockSpec((R, 8), lambda i: (i, 0)),
            pl.BlockSpec((D, D), lambda i: (0, 0)),
            pl.BlockSpec((1, D), lambda i: (0, 0)),
        ],
        out_specs=pl.BlockSpec((R, D), lambda i: (i, 0)),
        out_shape=jax.ShapeDtypeStruct((N, D), _f32),
    )(a0, a1, y1, dinv, w2p, b1r)


def _k3_body(a0_ref, a1_ref, y2_ref, dinv_ref, b2_ref, wn1_ref, bn1_ref,
             wn2_ref, bn2_ref, batch_ref, h2_ref, nlog_ref, sumh_ref,
             cnt_ref, maxn_ref):
    i = pl.program_id(0)
    dinv = dinv_ref[:, :1]
    acc = jnp.concatenate([a0_ref[...], a1_ref[...]], axis=1)
    h2 = jnp.maximum(dinv * (acc + y2_ref[...]) + b2_ref[...], 0.0)
    h2_ref[...] = h2
    hn = jnp.maximum(
        jnp.dot(h2, wn1_ref[...], preferred_element_type=_f32) + bn1_ref[...],
        0.0)
    nlog = jnp.sum(hn * wn2_ref[...], axis=1, keepdims=True) + bn2_ref[0, :1]
    nlog_ref[...] = jnp.broadcast_to(nlog, (R, 8))
    oh = (batch_ref[...] == lax.broadcasted_iota(jnp.int32, (R, G), 1)
          ).astype(_f32)
    psumh = lax.dot_general(oh, h2, (((0,), (0,)), ((), ())),
                            preferred_element_type=_f32)
    pcnt = jnp.sum(oh, axis=0)[:, None]
    pmax = jnp.max(jnp.where(oh > 0, nlog, -1e30), axis=0)[:, None]

    @pl.when(i == 0)
    def _():
        sumh_ref[...] = jnp.zeros_like(sumh_ref)
        cnt_ref[...] = jnp.zeros_like(cnt_ref)
        maxn_ref[...] = jnp.full_like(maxn_ref, -1e30)

    sumh_ref[...] += psumh
    cnt_ref[...] += jnp.broadcast_to(pcnt, (G, 8))
    maxn_ref[...] = jnp.maximum(maxn_ref[...], jnp.broadcast_to(pmax, (G, 8)))


def _k3(a0, a1, y2, dinv, b2r, wn1p, bn1r, wn2r, bn2r, batch2d):
    return pl.pallas_call(
        _k3_body,
        grid=(NBLK,),
        in_specs=[
            pl.BlockSpec((R, HALF), lambda i: (i, 0)),
            pl.BlockSpec((R, HALF), lambda i: (i, 0)),
            pl.BlockSpec((R, D), lambda i: (i, 0)),
            pl.BlockSpec((R, 8), lambda i: (i, 0)),
            pl.BlockSpec((1, D), lambda i: (0, 0)),
            pl.BlockSpec((D, 16), lambda i: (0, 0)),
            pl.BlockSpec((1, 16), lambda i: (0, 0)),
            pl.BlockSpec((1, 16), lambda i: (0, 0)),
            pl.BlockSpec((1, 8), lambda i: (0, 0)),
            pl.BlockSpec((R, 1), lambda i: (i, 0)),
        ],
        out_specs=[
            pl.BlockSpec((R, D), lambda i: (i, 0)),
            pl.BlockSpec((R, 8), lambda i: (i, 0)),
            pl.BlockSpec((G, D), lambda i: (0, 0)),
            pl.BlockSpec((G, 8), lambda i: (0, 0)),
            pl.BlockSpec((G, 8), lambda i: (0, 0)),
        ],
        out_shape=[
            jax.ShapeDtypeStruct((N, D), _f32),
            jax.ShapeDtypeStruct((N, 8), _f32),
            jax.ShapeDtypeStruct((G, D), _f32),
            jax.ShapeDtypeStruct((G, 8), _f32),
            jax.ShapeDtypeStruct((G, 8), _f32),
        ],
        compiler_params=pltpu.CompilerParams(
            dimension_semantics=("arbitrary",)),
    )(a0, a1, y2, dinv, b2r, wn1p, bn1r, wn2r, bn2r, batch2d)


def _k5_body(nlog_ref, h2_ref, batch_ref, maxn_ref, e_ref, sume_ref,
             sumeh_ref):
    i = pl.program_id(0)
    oh = (batch_ref[...] == lax.broadcasted_iota(jnp.int32, (R, G), 1)
          ).astype(_f32)
    mrow = lax.dot_general(oh, maxn_ref[:, :1], (((1,), (0,)), ((), ())),
                           preferred_element_type=_f32)
    e = jnp.exp(nlog_ref[:, :1] - mrow)
    e_ref[...] = jnp.broadcast_to(e, (R, 8))
    psume = lax.dot_general(oh, e, (((0,), (0,)), ((), ())),
                            preferred_element_type=_f32)
    psumeh = lax.dot_general(oh, e * h2_ref[...], (((0,), (0,)), ((), ())),
                             preferred_element_type=_f32)

    @pl.when(i == 0)
    def _():
        sume_ref[...] = jnp.zeros_like(sume_ref)
        sumeh_ref[...] = jnp.zeros_like(sumeh_ref)

    sume_ref[...] += jnp.broadcast_to(psume, (G, 8))
    sumeh_ref[...] += psumeh


def _k5(nlog, h2, batch2d, maxn):
    return pl.pallas_call(
        _k5_body,
        grid=(NBLK,),
        in_specs=[
            pl.BlockSpec((R, 8), lambda i: (i, 0)),
            pl.BlockSpec((R, D), lambda i: (i, 0)),
            pl.BlockSpec((R, 1), lambda i: (i, 0)),
            pl.BlockSpec((G, 8), lambda i: (0, 0)),
        ],
        out_specs=[
            pl.BlockSpec((R, 8), lambda i: (i, 0)),
            pl.BlockSpec((G, 8), lambda i: (0, 0)),
            pl.BlockSpec((G, D), lambda i: (0, 0)),
        ],
        out_shape=[
            jax.ShapeDtypeStruct((N, 8), _f32),
            jax.ShapeDtypeStruct((G, 8), _f32),
            jax.ShapeDtypeStruct((G, D), _f32),
        ],
        compiler_params=pltpu.CompilerParams(
            dimension_semantics=("arbitrary",)),
    )(nlog, h2, batch2d, maxn)


def _k6_body(e_ref, batch_ref, sume_ref, n_ref):
    oh = (batch_ref[...] == lax.broadcasted_iota(jnp.int32, (R, G), 1)
          ).astype(_f32)
    rinv = 1.0 / jnp.maximum(sume_ref[:, :1], 1e-30)
    rrow = lax.dot_general(oh, rinv, (((1,), (0,)), ((), ())),
                           preferred_element_type=_f32)
    n_ref[...] = e_ref[:, :1] * rrow


def _k6(e, batch2d, sume):
    return pl.pallas_call(
        _k6_body,
        grid=(NBLK,),
        in_specs=[
            pl.BlockSpec((R, 8), lambda i: (i, 0)),
            pl.BlockSpec((R, 1), lambda i: (i, 0)),
            pl.BlockSpec((G, 8), lambda i: (0, 0)),
        ],
        out_specs=pl.BlockSpec((R, 1), lambda i: (i, 0)),
        out_shape=jax.ShapeDtypeStruct((N, 1), _f32),
    )(e, batch2d, sume)


def _k7_body(sumh_ref, cnt_ref, sumeh_ref, sume_ref, wg_ref, bg_ref,
             wt_ref, bt_ref, wb1_ref, bb1_ref, wb2_ref, bb2_ref,
             t_ref, bout_ref):
    gm = sumh_ref[...] / jnp.maximum(cnt_ref[:, :1], 1.0)
    g = jnp.dot(gm, wg_ref[...], preferred_element_type=_f32) + bg_ref[...]
    logit = jnp.dot(g, wt_ref[...], preferred_element_type=_f32) + bt_ref[...]
    m = jnp.max(logit, axis=1, keepdims=True)
    ez = jnp.exp(logit - m)
    t_ref[...] = (ez / jnp.sum(ez, axis=1, keepdims=True))[:, :2]

    rinv = 1.0 / jnp.maximum(sume_ref[:, :1], 1e-30)
    bpool = sumeh_ref[...] * rinv
    hb = jnp.maximum(
        jnp.dot(bpool, wb1_ref[...], preferred_element_type=_f32)
        + bb1_ref[...], 0.0)
    z = jnp.dot(hb, wb2_ref[...], preferred_element_type=_f32) + bb2_ref[...]
    mz = jnp.max(z, axis=0, keepdims=True)
    ezz = jnp.exp(z - mz)
    bout_ref[...] = (ezz / jnp.sum(ezz, axis=0, keepdims=True))[:, :3]


def _k7(sumh, cnt, sumeh, sume, wgp, bgr, wtp, btp, wb1p, bb1r, wb2p, bb2r):
    def full(s):
        return pl.BlockSpec(s, lambda: (0, 0))
    return pl.pallas_call(
        _k7_body,
        grid=(),
        in_specs=[
            full((G, D)), full((G, 8)), full((G, D)), full((G, 8)),
            full((D, D)), full((1, D)), full((D, 8)), full((1, 8)),
            full((D, 16)), full((1, 16)), full((16, 8)), full((1, 8)),
        ],
        out_specs=[full((G, 2)), full((G, 3))],
        out_shape=[
            jax.ShapeDtypeStruct((G, 2), _f32),
            jax.ShapeDtypeStruct((G, 3), _f32),
        ],
    )(sumh, cnt, sumeh, sume, wgp, bgr, wtp, btp, wb1p, bb1r, wb2p, bb2r)


def kernel(x, edge_index, batch, W1, b1, W2, b2, Wg, bg, Wt, bt,
           Wn1, bn1, Wn2, bn2, Wb1, bb1, Wb2, bb2):
    src = edge_index[0]
    dst = edge_index[1]
    batch2d = batch.reshape(N, 1)

    W2p = jnp.pad(W2, ((0, 0), (0, D - 24)))
    b1r = b1.reshape(1, D)
    b2r = jnp.pad(b2, (0, D - 24)).reshape(1, D)
    Wn1p = jnp.pad(Wn1, ((0, D - 24), (0, 0)))
    bn1r = bn1.reshape(1, 16)
    wn2r = Wn2.reshape(1, 16)
    bn2r = jnp.pad(bn2, (0, 7)).reshape(1, 8)
    Wgp = jnp.pad(Wg, ((0, D - 24), (0, 0)))
    bgr = bg.reshape(1, D)
    Wtp = jnp.pad(Wt, ((0, 0), (0, 6)))
    btp = jnp.pad(bt, (0, 6), constant_values=-1e30).reshape(1, 8)
    Wb1p = jnp.pad(Wb1, ((0, D - 24), (0, 0)))
    bb1r = bb1.reshape(1, 16)
    Wb2p = jnp.pad(Wb2, ((0, 0), (0, 5)))
    bb2r = jnp.pad(bb2, (0, 5)).reshape(1, 8)

    ones_ch = jnp.ones((CH, HALF), _f32)
    zeros_rps = jnp.zeros((RPS, HALF), _f32)

    hist = _deg_hist(dst, ones_ch, zeros_rps)
    h0 = hist[:N]
    h1 = hist[N:]

    y1, dinv = _m1(x, h0, h1, W1)

    acc1 = _edge_pass(y1.reshape(2 * N, HALF), src, dst, zeros_rps)
    y2 = _k2(acc1[:N], acc1[N:], y1, dinv, W2p, b1r)

    acc2 = _edge_pass(y2.reshape(2 * N, HALF), src, dst, zeros_rps)
    h2, nlog, sumh, cnt, maxn = _k3(acc2[:N], acc2[N:], y2, dinv, b2r,
                                    Wn1p, bn1r, wn2r, bn2r, batch2d)

    e, sume, sumeh = _k5(nlog, h2, batch2d, maxn)
    n = _k6(e, batch2d, sume)
    t, bout = _k7(sumh, cnt, sumeh, sume, Wgp, bgr, Wtp, btp,
                  Wb1p, bb1r, Wb2p, bb2r)
    return (t, n, bout)


# speedup vs baseline: 10.7980x; 10.7980x over previous
import functools

import jax
import jax.numpy as jnp
from jax import lax
from jax.experimental import pallas as pl
from jax.experimental.pallas import tpu as pltpu
from jax.experimental.pallas import tpu_sc as plsc

N = 100000
E = 1600000
G = 256
D = 32
HALF = 16
NC = 2
NS = 16
NW = NC * NS
CH = 128
NCHUNK = E // CH
NPAD = 100096
RPS = NPAD // NS

R = 2000
NBLK = N // R

_f32 = jnp.float32


def _sc_mesh():
    return plsc.VectorSubcoreMesh(core_axis_name="c", subcore_axis_name="s")


def _deg_body(dst_hbm, ones_hbm, zeros_hbm, hist_hbm, dst_v, ones_v, hist_sh):
    c = lax.axis_index("c")
    s = lax.axis_index("s")
    w = s * NC + c
    pltpu.sync_copy(zeros_hbm, hist_sh.at[pl.ds(s * RPS, RPS)])
    pltpu.sync_copy(ones_hbm, ones_v)
    plsc.subcore_barrier()

    def body(i, carry):
        k = w + i * NW

        @pl.when(k < NCHUNK)
        def _():
            pltpu.sync_copy(dst_hbm.at[pl.ds(k * CH, CH)], dst_v)
            pltpu.sync_copy(ones_v, hist_sh.at[dst_v], add=True)

        return carry

    lax.fori_loop(0, (NCHUNK + NW - 1) // NW, body, 0)
    plsc.subcore_barrier()
    pltpu.sync_copy(hist_sh.at[pl.ds(s * RPS, RPS)],
                    hist_hbm.at[pl.ds(c * NPAD + s * RPS, RPS)])


def _deg_hist(dst, ones_ch, zeros_rps):
    return pl.kernel(
        _deg_body,
        out_type=jax.ShapeDtypeStruct((NC * NPAD, HALF), _f32),
        mesh=_sc_mesh(),
        compiler_params=pltpu.CompilerParams(use_tc_tiling_on_sc=False),
        scratch_types=[
            pltpu.VMEM((CH,), jnp.int32),
            pltpu.VMEM((CH, HALF), _f32),
            pltpu.VMEM_SHARED((NPAD, HALF), _f32),
        ],
    )(dst, ones_ch, zeros_rps)


def _conv_body(yflat_hbm, src_hbm, dst_hbm, zeros_hbm, acc_hbm,
               src_v, dst_v, gidx_v, rows_v, sem, acc_sh):
    c = lax.axis_index("c")
    s = lax.axis_index("s")
    pltpu.sync_copy(zeros_hbm, acc_sh.at[pl.ds(s * RPS, RPS)])
    plsc.subcore_barrier()

    def body(i, carry):
        k = s + i * NS

        @pl.when(k < NCHUNK)
        def _():
            e0 = k * CH
            pltpu.sync_copy(src_hbm.at[pl.ds(e0, CH)], src_v)
            pltpu.sync_copy(dst_hbm.at[pl.ds(e0, CH)], dst_v)
            for j in range(CH // 16):
                sl = pl.ds(j * 16, 16)
                gidx_v[sl] = src_v[sl] * 2 + c
            pltpu.async_copy(yflat_hbm.at[gidx_v], rows_v, sem).wait()
            pltpu.sync_copy(rows_v, acc_sh.at[dst_v], add=True)

        return carry

    lax.fori_loop(0, (NCHUNK + NS - 1) // NS, body, 0)
    plsc.subcore_barrier()
    pltpu.sync_copy(acc_sh.at[pl.ds(s * RPS, RPS)],
                    acc_hbm.at[pl.ds(c * NPAD + s * RPS, RPS)])


def _edge_pass(yflat, src, dst, zeros_rps):
    return pl.kernel(
        _conv_body,
        out_type=jax.ShapeDtypeStruct((NC * NPAD, HALF), _f32),
        mesh=_sc_mesh(),
        compiler_params=pltpu.CompilerParams(use_tc_tiling_on_sc=False),
        scratch_types=[
            pltpu.VMEM((CH,), jnp.int32),
            pltpu.VMEM((CH,), jnp.int32),
            pltpu.VMEM((CH,), jnp.int32),
            pltpu.VMEM((CH, HALF), _f32),
            pltpu.SemaphoreType.DMA,
            pltpu.VMEM_SHARED((NPAD, HALF), _f32),
        ],
    )(yflat, src, dst, zeros_rps)


def _m1_body(x_ref, h0_ref, h1_ref, w1_ref, y1_ref, dinv_ref):
    deg = 1.0 + h0_ref[:, :1] + h1_ref[:, :1]
    dinv = lax.rsqrt(deg)
    xw = jnp.dot(x_ref[...], w1_ref[...], preferred_element_type=_f32)
    y1_ref[...] = xw * dinv
    dinv_ref[...] = jnp.broadcast_to(dinv, (R, 8))


def _m1(x, h0, h1, w1):
    return pl.pallas_call(
        _m1_body,
        grid=(NBLK,),
        in_specs=[
            pl.BlockSpec((R, 128), lambda i: (i, 0)),
            pl.BlockSpec((R, HALF), lambda i: (i, 0)),
            pl.BlockSpec((R, HALF), lambda i: (i, 0)),
            pl.BlockSpec((128, D), lambda i: (0, 0)),
        ],
        out_specs=[
            pl.BlockSpec((R, D), lambda i: (i, 0)),
            pl.BlockSpec((R, 8), lambda i: (i, 0)),
        ],
        out_shape=[
            jax.ShapeDtypeStruct((N, D), _f32),
            jax.ShapeDtypeStruct((N, 8), _f32),
        ],
    )(x, h0, h1, w1)


def _k2_body(a0_ref, a1_ref, y1_ref, dinv_ref, w2_ref, b1_ref, y2_ref):
    dinv = dinv_ref[:, :1]
    acc = jnp.concatenate([a0_ref[...], a1_ref[...]], axis=1)
    h1 = jnp.maximum(dinv * (acc + y1_ref[...]) + b1_ref[...], 0.0)
    y2_ref[...] = jnp.dot(h1, w2_ref[...], preferred_element_type=_f32) * dinv


def _k2(a0, a1, y1, dinv, w2p, b1r):
    return pl.pallas_call(
        _k2_body,
        grid=(NBLK,),
        in_specs=[
            pl.BlockSpec((R, HALF), lambda i: (i, 0)),
            pl.BlockSpec((R, HALF), lambda i: (i, 0)),
            pl.BlockSpec((R, D), lambda i: (i, 0)),
            pl.BlockSpec((R, 8), lambda i: (i, 0)),
            pl.BlockSpec((D, D), lambda i: (0, 0)),
            pl.BlockSpec((1, D), lambda i: (0, 0)),
        ],
        out_specs=pl.BlockSpec((R, D), lambda i: (i, 0)),
        out_shape=jax.ShapeDtypeStruct((N, D), _f32),
    )(a0, a1, y1, dinv, w2p, b1r)


def _k3_body(a0_ref, a1_ref, y2_ref, dinv_ref, b2_ref, wn1_ref, bn1_ref,
             wn2_ref, bn2_ref, batch_ref, h2_ref, nlog_ref, sumh_ref,
             cnt_ref, maxn_ref):
    i = pl.program_id(0)
    dinv = dinv_ref[:, :1]
    acc = jnp.concatenate([a0_ref[...], a1_ref[...]], axis=1)
    h2 = jnp.maximum(dinv * (acc + y2_ref[...]) + b2_ref[...], 0.0)
    h2_ref[...] = h2
    hn = jnp.maximum(
        jnp.dot(h2, wn1_ref[...], preferred_element_type=_f32) + bn1_ref[...],
        0.0)
    nlog = jnp.sum(hn * wn2_ref[...], axis=1, keepdims=True) + bn2_ref[0, :1]
    nlog_ref[...] = jnp.broadcast_to(nlog, (R, 8))
    oh = (batch_ref[...] == lax.broadcasted_iota(jnp.int32, (R, G), 1)
          ).astype(_f32)
    psumh = lax.dot_general(oh, h2, (((0,), (0,)), ((), ())),
                            preferred_element_type=_f32)
    pcnt = jnp.sum(oh, axis=0)[:, None]
    pmax = jnp.max(jnp.where(oh > 0, nlog, -1e30), axis=0)[:, None]

    @pl.when(i == 0)
    def _():
        sumh_ref[...] = jnp.zeros_like(sumh_ref)
        cnt_ref[...] = jnp.zeros_like(cnt_ref)
        maxn_ref[...] = jnp.full_like(maxn_ref, -1e30)

    sumh_ref[...] += psumh
    cnt_ref[...] += jnp.broadcast_to(pcnt, (G, 8))
    maxn_ref[...] = jnp.maximum(maxn_ref[...], jnp.broadcast_to(pmax, (G, 8)))


def _k3(a0, a1, y2, dinv, b2r, wn1p, bn1r, wn2r, bn2r, batch2d):
    return pl.pallas_call(
        _k3_body,
        grid=(NBLK,),
        in_specs=[
            pl.BlockSpec((R, HALF), lambda i: (i, 0)),
            pl.BlockSpec((R, HALF), lambda i: (i, 0)),
            pl.BlockSpec((R, D), lambda i: (i, 0)),
            pl.BlockSpec((R, 8), lambda i: (i, 0)),
            pl.BlockSpec((1, D), lambda i: (0, 0)),
            pl.BlockSpec((D, 16), lambda i: (0, 0)),
            pl.BlockSpec((1, 16), lambda i: (0, 0)),
            pl.BlockSpec((1, 16), lambda i: (0, 0)),
            pl.BlockSpec((1, 8), lambda i: (0, 0)),
            pl.BlockSpec((R, 1), lambda i: (i, 0)),
        ],
        out_specs=[
            pl.BlockSpec((R, D), lambda i: (i, 0)),
            pl.BlockSpec((R, 8), lambda i: (i, 0)),
            pl.BlockSpec((G, D), lambda i: (0, 0)),
            pl.BlockSpec((G, 8), lambda i: (0, 0)),
            pl.BlockSpec((G, 8), lambda i: (0, 0)),
        ],
        out_shape=[
            jax.ShapeDtypeStruct((N, D), _f32),
            jax.ShapeDtypeStruct((N, 8), _f32),
            jax.ShapeDtypeStruct((G, D), _f32),
            jax.ShapeDtypeStruct((G, 8), _f32),
            jax.ShapeDtypeStruct((G, 8), _f32),
        ],
        compiler_params=pltpu.CompilerParams(
            dimension_semantics=("arbitrary",)),
    )(a0, a1, y2, dinv, b2r, wn1p, bn1r, wn2r, bn2r, batch2d)


def _k5_body(nlog_ref, h2_ref, batch_ref, maxn_ref, e_ref, sume_ref,
             sumeh_ref):
    i = pl.program_id(0)
    oh = (batch_ref[...] == lax.broadcasted_iota(jnp.int32, (R, G), 1)
          ).astype(_f32)
    mrow = lax.dot_general(oh, maxn_ref[:, :1], (((1,), (0,)), ((), ())),
                           preferred_element_type=_f32)
    e = jnp.exp(nlog_ref[:, :1] - mrow)
    e_ref[...] = jnp.broadcast_to(e, (R, 8))
    psume = lax.dot_general(oh, e, (((0,), (0,)), ((), ())),
                            preferred_element_type=_f32)
    psumeh = lax.dot_general(oh, e * h2_ref[...], (((0,), (0,)), ((), ())),
                             preferred_element_type=_f32)

    @pl.when(i == 0)
    def _():
        sume_ref[...] = jnp.zeros_like(sume_ref)
        sumeh_ref[...] = jnp.zeros_like(sumeh_ref)

    sume_ref[...] += jnp.broadcast_to(psume, (G, 8))
    sumeh_ref[...] += psumeh


def _k5(nlog, h2, batch2d, maxn):
    return pl.pallas_call(
        _k5_body,
        grid=(NBLK,),
        in_specs=[
            pl.BlockSpec((R, 8), lambda i: (i, 0)),
            pl.BlockSpec((R, D), lambda i: (i, 0)),
            pl.BlockSpec((R, 1), lambda i: (i, 0)),
            pl.BlockSpec((G, 8), lambda i: (0, 0)),
        ],
        out_specs=[
            pl.BlockSpec((R, 8), lambda i: (i, 0)),
            pl.BlockSpec((G, 8), lambda i: (0, 0)),
            pl.BlockSpec((G, D), lambda i: (0, 0)),
        ],
        out_shape=[
            jax.ShapeDtypeStruct((N, 8), _f32),
            jax.ShapeDtypeStruct((G, 8), _f32),
            jax.ShapeDtypeStruct((G, D), _f32),
        ],
        compiler_params=pltpu.CompilerParams(
            dimension_semantics=("arbitrary",)),
    )(nlog, h2, batch2d, maxn)


def _k6_body(e_ref, batch_ref, sume_ref, n_ref):
    oh = (batch_ref[...] == lax.broadcasted_iota(jnp.int32, (R, G), 1)
          ).astype(_f32)
    rinv = 1.0 / jnp.maximum(sume_ref[:, :1], 1e-30)
    rrow = lax.dot_general(oh, rinv, (((1,), (0,)), ((), ())),
                           preferred_element_type=_f32)
    n_ref[...] = e_ref[:, :1] * rrow


def _k6(e, batch2d, sume):
    return pl.pallas_call(
        _k6_body,
        grid=(NBLK,),
        in_specs=[
            pl.BlockSpec((R, 8), lambda i: (i, 0)),
            pl.BlockSpec((R, 1), lambda i: (i, 0)),
            pl.BlockSpec((G, 8), lambda i: (0, 0)),
        ],
        out_specs=pl.BlockSpec((R, 1), lambda i: (i, 0)),
        out_shape=jax.ShapeDtypeStruct((N, 1), _f32),
    )(e, batch2d, sume)


def _k7_body(sumh_ref, cnt_ref, sumeh_ref, sume_ref, wg_ref, bg_ref,
             wt_ref, bt_ref, wb1_ref, bb1_ref, wb2_ref, bb2_ref,
             t_ref, bout_ref):
    gm = sumh_ref[...] / jnp.maximum(cnt_ref[:, :1], 1.0)
    g = jnp.dot(gm, wg_ref[...], preferred_element_type=_f32) + bg_ref[...]
    logit = jnp.dot(g, wt_ref[...], preferred_element_type=_f32) + bt_ref[...]
    m = jnp.max(logit, axis=1, keepdims=True)
    ez = jnp.exp(logit - m)
    t_ref[...] = (ez / jnp.sum(ez, axis=1, keepdims=True))[:, :2]

    rinv = 1.0 / jnp.maximum(sume_ref[:, :1], 1e-30)
    bpool = sumeh_ref[...] * rinv
    hb = jnp.maximum(
        jnp.dot(bpool, wb1_ref[...], preferred_element_type=_f32)
        + bb1_ref[...], 0.0)
    z = jnp.dot(hb, wb2_ref[...], preferred_element_type=_f32) + bb2_ref[...]
    mz = jnp.max(z, axis=0, keepdims=True)
    ezz = jnp.exp(z - mz)
    bout_ref[...] = (ezz / jnp.sum(ezz, axis=0, keepdims=True))[:, :3]


def _k7(sumh, cnt, sumeh, sume, wgp, bgr, wtp, btp, wb1p, bb1r, wb2p, bb2r):
    def full(s):
        return pl.BlockSpec(s, lambda: (0, 0))
    return pl.pallas_call(
        _k7_body,
        grid=(),
        in_specs=[
            full((G, D)), full((G, 8)), full((G, D)), full((G, 8)),
            full((D, D)), full((1, D)), full((D, 8)), full((1, 8)),
            full((D, 16)), full((1, 16)), full((16, 8)), full((1, 8)),
        ],
        out_specs=[full((G, 2)), full((G, 3))],
        out_shape=[
            jax.ShapeDtypeStruct((G, 2), _f32),
            jax.ShapeDtypeStruct((G, 3), _f32),
        ],
    )(sumh, cnt, sumeh, sume, wgp, bgr, wtp, btp, wb1p, bb1r, wb2p, bb2r)


def kernel(x, edge_index, batch, W1, b1, W2, b2, Wg, bg, Wt, bt,
           Wn1, bn1, Wn2, bn2, Wb1, bb1, Wb2, bb2):
    src = edge_index[0]
    dst = edge_index[1]
    batch2d = batch.reshape(N, 1)

    W2p = jnp.pad(W2, ((0, 0), (0, D - 24)))
    b1r = b1.reshape(1, D)
    b2r = jnp.pad(b2, (0, D - 24)).reshape(1, D)
    Wn1p = jnp.pad(Wn1, ((0, D - 24), (0, 0)))
    bn1r = bn1.reshape(1, 16)
    wn2r = Wn2.reshape(1, 16)
    bn2r = jnp.pad(bn2, (0, 7)).reshape(1, 8)
    Wgp = jnp.pad(Wg, ((0, D - 24), (0, 0)))
    bgr = bg.reshape(1, D)
    Wtp = jnp.pad(Wt, ((0, 0), (0, 6)))
    btp = jnp.pad(bt, (0, 6), constant_values=-1e30).reshape(1, 8)
    Wb1p = jnp.pad(Wb1, ((0, D - 24), (0, 0)))
    bb1r = bb1.reshape(1, 16)
    Wb2p = jnp.pad(Wb2, ((0, 0), (0, 5)))
    bb2r = jnp.pad(bb2, (0, 5)).reshape(1, 8)

    ones_ch = jnp.ones((CH, HALF), _f32)
    zeros_rps = jnp.zeros((RPS, HALF), _f32)

    hist = _deg_hist(dst, ones_ch, zeros_rps)
    h0 = hist[:N]
    h1 = hist[NPAD:NPAD + N]

    y1, dinv = _m1(x, h0, h1, W1)

    acc1 = _edge_pass(y1.reshape(2 * N, HALF), src, dst, zeros_rps)
    y2 = _k2(acc1[:N], acc1[NPAD:NPAD + N], y1, dinv, W2p, b1r)

    acc2 = _edge_pass(y2.reshape(2 * N, HALF), src, dst, zeros_rps)
    h2, nlog, sumh, cnt, maxn = _k3(acc2[:N], acc2[NPAD:NPAD + N], y2, dinv, b2r,
                                    Wn1p, bn1r, wn2r, bn2r, batch2d)

    e, sume, sumeh = _k5(nlog, h2, batch2d, maxn)
    n = _k6(e, batch2d, sume)
    t, bout = _k7(sumh, cnt, sumeh, sume, Wgp, bgr, Wtp, btp,
                  Wb1p, bb1r, Wb2p, bb2r)
    return (t, n, bout)


# speedup vs baseline: 23.1466x; 2.1436x over previous
import functools

import jax
import jax.numpy as jnp
from jax import lax
from jax.experimental import pallas as pl
from jax.experimental.pallas import tpu as pltpu
from jax.experimental.pallas import tpu_sc as plsc

N = 100000
E = 1600000
G = 256
D = 32
HALF = 16
NC = 2
NS = 16
NW = NC * NS
CH = 128
NB = 8
NCHUNK = 12544
EPAD = NCHUNK * CH
GPS = NCHUNK // (NS * NB)
GPW = NCHUNK // (NS * NC * NB)
NPAD = 100096
RPS = NPAD // NS

R = 2000
NBLK = N // R

_f32 = jnp.float32


def _sc_mesh():
    return plsc.VectorSubcoreMesh(core_axis_name="c", subcore_axis_name="s")


def _deg_body(dst_hbm, ones_hbm, zeros_hbm, hist_hbm, dst_v, ones_v, sem,
              hist_sh):
    c = lax.axis_index("c")
    s = lax.axis_index("s")
    w = s * NC + c
    pltpu.sync_copy(zeros_hbm, hist_sh.at[pl.ds(s * RPS, RPS)])
    pltpu.sync_copy(ones_hbm, ones_v)
    plsc.subcore_barrier()

    def body(i, carry):
        row0 = (w * GPW + i) * NB
        pltpu.sync_copy(dst_hbm.at[pl.ds(row0, NB)], dst_v)
        descs = [pltpu.async_copy(ones_v, hist_sh.at[dst_v.at[b]], sem,
                                  add=True)
                 for b in range(NB)]
        for d_ in descs:
            d_.wait()
        return carry

    lax.fori_loop(0, GPW, body, 0)
    plsc.subcore_barrier()
    pltpu.sync_copy(hist_sh.at[pl.ds(s * RPS, RPS)],
                    hist_hbm.at[pl.ds(c * NPAD + s * RPS, RPS)])


def _deg_hist(dst2d, ones_ch, zeros_rps):
    return pl.kernel(
        _deg_body,
        out_type=jax.ShapeDtypeStruct((NC * NPAD, HALF), _f32),
        mesh=_sc_mesh(),
        compiler_params=pltpu.CompilerParams(use_tc_tiling_on_sc=False),
        scratch_types=[
            pltpu.VMEM((NB, CH), jnp.int32),
            pltpu.VMEM((CH, HALF), _f32),
            pltpu.SemaphoreType.DMA,
            pltpu.VMEM_SHARED((NPAD, HALF), _f32),
        ],
    )(dst2d, ones_ch, zeros_rps)


def _conv_body(yflat_hbm, src_hbm, dst_hbm, zeros_hbm, acc_hbm,
               src_v, dst_v, gidx_v, rows_v, sem, sem2, acc_sh):
    c = lax.axis_index("c")
    s = lax.axis_index("s")
    pltpu.sync_copy(zeros_hbm, acc_sh.at[pl.ds(s * RPS, RPS)])
    plsc.subcore_barrier()

    def body(i, carry):
        row0 = (s * GPS + i) * NB
        pltpu.sync_copy(src_hbm.at[pl.ds(row0, NB)], src_v)
        pltpu.sync_copy(dst_hbm.at[pl.ds(row0, NB)], dst_v)
        for b in range(NB):
            for j in range(CH // 16):
                sl = pl.ds(j * 16, 16)
                gidx_v[b, sl] = src_v[b, sl] * 2 + c
        descs = [pltpu.async_copy(yflat_hbm.at[gidx_v.at[b]],
                                  rows_v.at[pl.ds(b * CH, CH)], sem)
                 for b in range(NB)]
        for d_ in descs:
            d_.wait()
        descs2 = [pltpu.async_copy(rows_v.at[pl.ds(b * CH, CH)],
                                   acc_sh.at[dst_v.at[b]], sem2, add=True)
                  for b in range(NB)]
        for d_ in descs2:
            d_.wait()
        return carry

    lax.fori_loop(0, GPS, body, 0)
    plsc.subcore_barrier()
    pltpu.sync_copy(acc_sh.at[pl.ds(s * RPS, RPS)],
                    acc_hbm.at[pl.ds(c * NPAD + s * RPS, RPS)])


def _edge_pass(yflat, src2d, dst2d, zeros_rps):
    return pl.kernel(
        _conv_body,
        out_type=jax.ShapeDtypeStruct((NC * NPAD, HALF), _f32),
        mesh=_sc_mesh(),
        compiler_params=pltpu.CompilerParams(use_tc_tiling_on_sc=False),
        scratch_types=[
            pltpu.VMEM((NB, CH), jnp.int32),
            pltpu.VMEM((NB, CH), jnp.int32),
            pltpu.VMEM((NB, CH), jnp.int32),
            pltpu.VMEM((NB * CH, HALF), _f32),
            pltpu.SemaphoreType.DMA,
            pltpu.SemaphoreType.DMA,
            pltpu.VMEM_SHARED((NPAD, HALF), _f32),
        ],
    )(yflat, src2d, dst2d, zeros_rps)


def _m1_body(x_ref, h0_ref, h1_ref, w1_ref, y1_ref, dinv_ref):
    deg = 1.0 + h0_ref[:, :1] + h1_ref[:, :1]
    dinv = lax.rsqrt(deg)
    xw = jnp.dot(x_ref[...], w1_ref[...], preferred_element_type=_f32)
    y1_ref[...] = xw * dinv
    dinv_ref[...] = jnp.broadcast_to(dinv, (R, 8))


def _m1(x, h0, h1, w1):
    return pl.pallas_call(
        _m1_body,
        grid=(NBLK,),
        in_specs=[
            pl.BlockSpec((R, 128), lambda i: (i, 0)),
            pl.BlockSpec((R, HALF), lambda i: (i, 0)),
            pl.BlockSpec((R, HALF), lambda i: (i, 0)),
            pl.BlockSpec((128, D), lambda i: (0, 0)),
        ],
        out_specs=[
            pl.BlockSpec((R, D), lambda i: (i, 0)),
            pl.BlockSpec((R, 8), lambda i: (i, 0)),
        ],
        out_shape=[
            jax.ShapeDtypeStruct((N, D), _f32),
            jax.ShapeDtypeStruct((N, 8), _f32),
        ],
    )(x, h0, h1, w1)


def _k2_body(a0_ref, a1_ref, y1_ref, dinv_ref, w2_ref, b1_ref, y2_ref):
    dinv = dinv_ref[:, :1]
    acc = jnp.concatenate([a0_ref[...], a1_ref[...]], axis=1)
    h1 = jnp.maximum(dinv * (acc + y1_ref[...]) + b1_ref[...], 0.0)
    y2_ref[...] = jnp.dot(h1, w2_ref[...], preferred_element_type=_f32) * dinv


def _k2(a0, a1, y1, dinv, w2p, b1r):
    return pl.pallas_call(
        _k2_body,
        grid=(NBLK,),
        in_specs=[
            pl.BlockSpec((R, HALF), lambda i: (i, 0)),
            pl.BlockSpec((R, HALF), lambda i: (i, 0)),
            pl.BlockSpec((R, D), lambda i: (i, 0)),
            pl.BlockSpec((R, 8), lambda i: (i, 0)),
            pl.BlockSpec((D, D), lambda i: (0, 0)),
            pl.BlockSpec((1, D), lambda i: (0, 0)),
        ],
        out_specs=pl.BlockSpec((R, D), lambda i: (i, 0)),
        out_shape=jax.ShapeDtypeStruct((N, D), _f32),
    )(a0, a1, y1, dinv, w2p, b1r)


def _k3_body(a0_ref, a1_ref, y2_ref, dinv_ref, b2_ref, wn1_ref, bn1_ref,
             wn2_ref, bn2_ref, batch_ref, h2_ref, nlog_ref, sumh_ref,
             cnt_ref, maxn_ref):
    i = pl.program_id(0)
    dinv = dinv_ref[:, :1]
    acc = jnp.concatenate([a0_ref[...], a1_ref[...]], axis=1)
    h2 = jnp.maximum(dinv * (acc + y2_ref[...]) + b2_ref[...], 0.0)
    h2_ref[...] = h2
    hn = jnp.maximum(
        jnp.dot(h2, wn1_ref[...], preferred_element_type=_f32) + bn1_ref[...],
        0.0)
    nlog = jnp.sum(hn * wn2_ref[...], axis=1, keepdims=True) + bn2_ref[0, :1]
    nlog_ref[...] = jnp.broadcast_to(nlog, (R, 8))
    oh = (batch_ref[...] == lax.broadcasted_iota(jnp.int32, (R, G), 1)
          ).astype(_f32)
    psumh = lax.dot_general(oh, h2, (((0,), (0,)), ((), ())),
                            preferred_element_type=_f32)
    pcnt = jnp.sum(oh, axis=0)[:, None]
    pmax = jnp.max(jnp.where(oh > 0, nlog, -1e30), axis=0)[:, None]

    @pl.when(i == 0)
    def _():
        sumh_ref[...] = jnp.zeros_like(sumh_ref)
        cnt_ref[...] = jnp.zeros_like(cnt_ref)
        maxn_ref[...] = jnp.full_like(maxn_ref, -1e30)

    sumh_ref[...] += psumh
    cnt_ref[...] += jnp.broadcast_to(pcnt, (G, 8))
    maxn_ref[...] = jnp.maximum(maxn_ref[...], jnp.broadcast_to(pmax, (G, 8)))


def _k3(a0, a1, y2, dinv, b2r, wn1p, bn1r, wn2r, bn2r, batch2d):
    return pl.pallas_call(
        _k3_body,
        grid=(NBLK,),
        in_specs=[
            pl.BlockSpec((R, HALF), lambda i: (i, 0)),
            pl.BlockSpec((R, HALF), lambda i: (i, 0)),
            pl.BlockSpec((R, D), lambda i: (i, 0)),
            pl.BlockSpec((R, 8), lambda i: (i, 0)),
            pl.BlockSpec((1, D), lambda i: (0, 0)),
            pl.BlockSpec((D, 16), lambda i: (0, 0)),
            pl.BlockSpec((1, 16), lambda i: (0, 0)),
            pl.BlockSpec((1, 16), lambda i: (0, 0)),
            pl.BlockSpec((1, 8), lambda i: (0, 0)),
            pl.BlockSpec((R, 1), lambda i: (i, 0)),
        ],
        out_specs=[
            pl.BlockSpec((R, D), lambda i: (i, 0)),
            pl.BlockSpec((R, 8), lambda i: (i, 0)),
            pl.BlockSpec((G, D), lambda i: (0, 0)),
            pl.BlockSpec((G, 8), lambda i: (0, 0)),
            pl.BlockSpec((G, 8), lambda i: (0, 0)),
        ],
        out_shape=[
            jax.ShapeDtypeStruct((N, D), _f32),
            jax.ShapeDtypeStruct((N, 8), _f32),
            jax.ShapeDtypeStruct((G, D), _f32),
            jax.ShapeDtypeStruct((G, 8), _f32),
            jax.ShapeDtypeStruct((G, 8), _f32),
        ],
        compiler_params=pltpu.CompilerParams(
            dimension_semantics=("arbitrary",)),
    )(a0, a1, y2, dinv, b2r, wn1p, bn1r, wn2r, bn2r, batch2d)


def _k5_body(nlog_ref, h2_ref, batch_ref, maxn_ref, e_ref, sume_ref,
             sumeh_ref):
    i = pl.program_id(0)
    oh = (batch_ref[...] == lax.broadcasted_iota(jnp.int32, (R, G), 1)
          ).astype(_f32)
    mrow = lax.dot_general(oh, maxn_ref[:, :1], (((1,), (0,)), ((), ())),
                           preferred_element_type=_f32)
    e = jnp.exp(nlog_ref[:, :1] - mrow)
    e_ref[...] = jnp.broadcast_to(e, (R, 8))
    psume = lax.dot_general(oh, e, (((0,), (0,)), ((), ())),
                            preferred_element_type=_f32)
    psumeh = lax.dot_general(oh, e * h2_ref[...], (((0,), (0,)), ((), ())),
                             preferred_element_type=_f32)

    @pl.when(i == 0)
    def _():
        sume_ref[...] = jnp.zeros_like(sume_ref)
        sumeh_ref[...] = jnp.zeros_like(sumeh_ref)

    sume_ref[...] += jnp.broadcast_to(psume, (G, 8))
    sumeh_ref[...] += psumeh


def _k5(nlog, h2, batch2d, maxn):
    return pl.pallas_call(
        _k5_body,
        grid=(NBLK,),
        in_specs=[
            pl.BlockSpec((R, 8), lambda i: (i, 0)),
            pl.BlockSpec((R, D), lambda i: (i, 0)),
            pl.BlockSpec((R, 1), lambda i: (i, 0)),
            pl.BlockSpec((G, 8), lambda i: (0, 0)),
        ],
        out_specs=[
            pl.BlockSpec((R, 8), lambda i: (i, 0)),
            pl.BlockSpec((G, 8), lambda i: (0, 0)),
            pl.BlockSpec((G, D), lambda i: (0, 0)),
        ],
        out_shape=[
            jax.ShapeDtypeStruct((N, 8), _f32),
            jax.ShapeDtypeStruct((G, 8), _f32),
            jax.ShapeDtypeStruct((G, D), _f32),
        ],
        compiler_params=pltpu.CompilerParams(
            dimension_semantics=("arbitrary",)),
    )(nlog, h2, batch2d, maxn)


def _k6_body(e_ref, batch_ref, sume_ref, n_ref):
    oh = (batch_ref[...] == lax.broadcasted_iota(jnp.int32, (R, G), 1)
          ).astype(_f32)
    rinv = 1.0 / jnp.maximum(sume_ref[:, :1], 1e-30)
    rrow = lax.dot_general(oh, rinv, (((1,), (0,)), ((), ())),
                           preferred_element_type=_f32)
    n_ref[...] = e_ref[:, :1] * rrow


def _k6(e, batch2d, sume):
    return pl.pallas_call(
        _k6_body,
        grid=(NBLK,),
        in_specs=[
            pl.BlockSpec((R, 8), lambda i: (i, 0)),
            pl.BlockSpec((R, 1), lambda i: (i, 0)),
            pl.BlockSpec((G, 8), lambda i: (0, 0)),
        ],
        out_specs=pl.BlockSpec((R, 1), lambda i: (i, 0)),
        out_shape=jax.ShapeDtypeStruct((N, 1), _f32),
    )(e, batch2d, sume)


def _k7_body(sumh_ref, cnt_ref, sumeh_ref, sume_ref, wg_ref, bg_ref,
             wt_ref, bt_ref, wb1_ref, bb1_ref, wb2_ref, bb2_ref,
             t_ref, bout_ref):
    gm = sumh_ref[...] / jnp.maximum(cnt_ref[:, :1], 1.0)
    g = jnp.dot(gm, wg_ref[...], preferred_element_type=_f32) + bg_ref[...]
    logit = jnp.dot(g, wt_ref[...], preferred_element_type=_f32) + bt_ref[...]
    m = jnp.max(logit, axis=1, keepdims=True)
    ez = jnp.exp(logit - m)
    t_ref[...] = (ez / jnp.sum(ez, axis=1, keepdims=True))[:, :2]

    rinv = 1.0 / jnp.maximum(sume_ref[:, :1], 1e-30)
    bpool = sumeh_ref[...] * rinv
    hb = jnp.maximum(
        jnp.dot(bpool, wb1_ref[...], preferred_element_type=_f32)
        + bb1_ref[...], 0.0)
    z = jnp.dot(hb, wb2_ref[...], preferred_element_type=_f32) + bb2_ref[...]
    mz = jnp.max(z, axis=0, keepdims=True)
    ezz = jnp.exp(z - mz)
    bout_ref[...] = (ezz / jnp.sum(ezz, axis=0, keepdims=True))[:, :3]


def _k7(sumh, cnt, sumeh, sume, wgp, bgr, wtp, btp, wb1p, bb1r, wb2p, bb2r):
    def full(s):
        return pl.BlockSpec(s, lambda: (0, 0))
    return pl.pallas_call(
        _k7_body,
        grid=(),
        in_specs=[
            full((G, D)), full((G, 8)), full((G, D)), full((G, 8)),
            full((D, D)), full((1, D)), full((D, 8)), full((1, 8)),
            full((D, 16)), full((1, 16)), full((16, 8)), full((1, 8)),
        ],
        out_specs=[full((G, 2)), full((G, 3))],
        out_shape=[
            jax.ShapeDtypeStruct((G, 2), _f32),
            jax.ShapeDtypeStruct((G, 3), _f32),
        ],
    )(sumh, cnt, sumeh, sume, wgp, bgr, wtp, btp, wb1p, bb1r, wb2p, bb2r)


def kernel(x, edge_index, batch, W1, b1, W2, b2, Wg, bg, Wt, bt,
           Wn1, bn1, Wn2, bn2, Wb1, bb1, Wb2, bb2):
    src2d = jnp.concatenate(
        [edge_index[0], jnp.zeros((EPAD - E,), jnp.int32)]).reshape(NCHUNK, CH)
    dst2d = jnp.concatenate(
        [edge_index[1], jnp.full((EPAD - E,), N, jnp.int32)]).reshape(NCHUNK,
                                                                      CH)
    batch2d = batch.reshape(N, 1)

    W2p = jnp.pad(W2, ((0, 0), (0, D - 24)))
    b1r = b1.reshape(1, D)
    b2r = jnp.pad(b2, (0, D - 24)).reshape(1, D)
    Wn1p = jnp.pad(Wn1, ((0, D - 24), (0, 0)))
    bn1r = bn1.reshape(1, 16)
    wn2r = Wn2.reshape(1, 16)
    bn2r = jnp.pad(bn2, (0, 7)).reshape(1, 8)
    Wgp = jnp.pad(Wg, ((0, D - 24), (0, 0)))
    bgr = bg.reshape(1, D)
    Wtp = jnp.pad(Wt, ((0, 0), (0, 6)))
    btp = jnp.pad(bt, (0, 6), constant_values=-1e30).reshape(1, 8)
    Wb1p = jnp.pad(Wb1, ((0, D - 24), (0, 0)))
    bb1r = bb1.reshape(1, 16)
    Wb2p = jnp.pad(Wb2, ((0, 0), (0, 5)))
    bb2r = jnp.pad(bb2, (0, 5)).reshape(1, 8)

    ones_ch = jnp.ones((CH, HALF), _f32)
    zeros_rps = jnp.zeros((RPS, HALF), _f32)

    hist = _deg_hist(dst2d, ones_ch, zeros_rps)
    h0 = hist[:N]
    h1 = hist[NPAD:NPAD + N]

    y1, dinv = _m1(x, h0, h1, W1)

    acc1 = _edge_pass(y1.reshape(2 * N, HALF), src2d, dst2d, zeros_rps)
    y2 = _k2(acc1[:N], acc1[NPAD:NPAD + N], y1, dinv, W2p, b1r)

    acc2 = _edge_pass(y2.reshape(2 * N, HALF), src2d, dst2d, zeros_rps)
    h2, nlog, sumh, cnt, maxn = _k3(acc2[:N], acc2[NPAD:NPAD + N], y2, dinv, b2r,
                                    Wn1p, bn1r, wn2r, bn2r, batch2d)

    e, sume, sumeh = _k5(nlog, h2, batch2d, maxn)
    n = _k6(e, batch2d, sume)
    t, bout = _k7(sumh, cnt, sumeh, sume, Wgp, bgr, Wtp, btp,
                  Wb1p, bb1r, Wb2p, bb2r)
    return (t, n, bout)
